# baseline (speedup 1.0000x reference)
import functools

import numpy as np
import jax
import jax.numpy as jnp
from jax.experimental import pallas as pl
from jax.experimental.pallas import tpu as pltpu

B = 512
D = 64
NTAB = 256
NSC = 32


def _bounds_tail() -> np.ndarray:
    parts = [np.array([-np.inf, 0.0], dtype=np.float32)]
    for lo, hi in [(-15, -10), (-10, -5), (-5, 0), (0, 5), (5, 10), (10, 15)]:
        parts.append(np.logspace(lo, hi, 128 // 6).astype(np.float32))
    b = np.unique(np.concatenate(parts))
    with np.errstate(invalid="ignore"):
        bd = np.log1p(b).astype(np.float32)
    tail = bd[1:]
    out = np.full((1, 128), np.inf, dtype=np.float32)
    out[0, : tail.shape[0]] = tail
    return out


_BOUNDS = _bounds_tail()

_HIGH = jax.lax.Precision.HIGHEST


def _gelu(x):
    return 0.5 * x * (1.0 + jax.lax.erf(x * np.float32(1.0 / np.sqrt(2.0))))


def _ln(x, g, b, eps=1e-5):
    m = jnp.mean(x, axis=-1, keepdims=True)
    v = jnp.mean((x - m) * (x - m), axis=-1, keepdims=True)
    return (x - m) * jax.lax.rsqrt(v + eps) * g + b


def _kernel(number_ref, mag_table_ref, scale_table_ref, w1_ref, b1_ref, g1_ref,
            be1_ref, w2_ref, b2_ref, g2_ref, be2_ref, mag_scale_ref, temp_ref,
            bounds_ref, out_ref, m_s, s_s, f_s, *, block_i):
    @pl.when(pl.program_id(0) == 0)
    def prologue():
        num = number_ref[...]
        signs = jnp.sign(num)
        a = jnp.abs(num)
        log_abs = jnp.log1p(a + 1e-15)
        scale_factor = jnp.floor(jnp.log10(a + 1e-15))
        scale_idx = jnp.clip(scale_factor + 16.0, 0.0, 31.0).astype(jnp.int32)

        feats = jnp.concatenate([log_abs, signs, num, scale_factor], axis=1)
        h = jnp.dot(feats, w1_ref[...].T, precision=_HIGH) + b1_ref[...]
        h = _ln(h, g1_ref[...], be1_ref[...])
        h = _gelu(h)
        h = jnp.dot(h, w2_ref[...].T, precision=_HIGH) + b2_ref[...]
        h = _ln(h, g2_ref[...], be2_ref[...])
        numerical = _gelu(h)

        bin_idx = 1 + jnp.sum(
            (bounds_ref[...] < log_abs).astype(jnp.int32), axis=1, keepdims=True
        )

        cols_tab = jax.lax.broadcasted_iota(jnp.int32, (B, NTAB), 1)
        oh_tab = (bin_idx == cols_tab).astype(jnp.float32)
        mag = jnp.dot(oh_tab, mag_table_ref[...], precision=_HIGH)
        sc_raw = jnp.dot(oh_tab, mag_scale_ref[...], precision=_HIGH)

        cols_sc = jax.lax.broadcasted_iota(jnp.int32, (B, NSC), 1)
        oh_sc = (scale_idx == cols_sc).astype(jnp.float32)
        s = jnp.dot(oh_sc, scale_table_ref[...], precision=_HIGH)

        scale = jax.nn.softplus(sc_raw / temp_ref[...])

        m = mag + numerical
        m_s[...] = m
        s_s[...] = s

        gram = jnp.dot(s, m.T, precision=_HIGH)
        mm = jnp.sum(m * m, axis=1, keepdims=True)
        ss = jnp.sum(s * s, axis=1, keepdims=True)
        n2 = ss + 2.0 * gram + mm.T
        t = jnp.sqrt(jnp.maximum(n2, 0.0))
        sc_row = scale.T
        f_s[...] = sc_row / jnp.maximum(sc_row * t, 1e-12)

    i0 = pl.program_id(0) * block_i
    s_blk = s_s[pl.ds(i0, block_i), :][:, None, :]
    f_blk = f_s[pl.ds(i0, block_i), :][:, :, None]
    out_ref[...] = (m_s[...][None, :, :] + s_blk) * f_blk


@jax.jit
def kernel(number, mag_table, scale_table, W1, b1, g1, be1, W2, b2, g2, be2,
           mag_scale, temperature):
    block_i = 64
    grid = (B // block_i,)

    def full(shape):
        return pl.BlockSpec(shape, lambda i: (0,) * len(shape))

    in_specs = [
        full((B, 1)),
        full((NTAB, D)),
        full((NSC, D)),
        full((D, 4)),
        full((1, D)),
        full((1, D)),
        full((1, D)),
        full((D, D)),
        full((1, D)),
        full((1, D)),
        full((1, D)),
        full((NTAB, 1)),
        full((1, 1)),
        full((1, 128)),
    ]
    out = pl.pallas_call(
        functools.partial(_kernel, block_i=block_i),
        grid=grid,
        in_specs=in_specs,
        out_specs=pl.BlockSpec((block_i, B, D), lambda i: (i, 0, 0)),
        out_shape=jax.ShapeDtypeStruct((B, B, D), jnp.float32),
        scratch_shapes=[
            pltpu.VMEM((B, D), jnp.float32),
            pltpu.VMEM((B, D), jnp.float32),
            pltpu.VMEM((B, B), jnp.float32),
        ],
    )(
        number, mag_table, scale_table, W1,
        b1.reshape(1, D), g1.reshape(1, D), be1.reshape(1, D), W2,
        b2.reshape(1, D), g2.reshape(1, D), be2.reshape(1, D),
        mag_scale.reshape(NTAB, 1), temperature.reshape(1, 1),
        jnp.asarray(_BOUNDS),
    )
    return out
